# baseline (speedup 1.0000x reference)
import functools

import jax
import jax.numpy as jnp
from jax import lax
from jax.experimental import pallas as pl
from jax.experimental.pallas import tpu as pltpu
from jax.experimental.pallas import tpu_sc as plsc

B = 4
S = 2048
D = 2048
NC = 2
NS = 16
NW = NC * NS
PRW = S // NW
C = 4
NCHUNK = PRW // C
CD = C * D
L = 16
UNROLL = 8
NBUF = 3

_mesh = plsc.VectorSubcoreMesh(core_axis_name="c", subcore_axis_name="s")


@functools.partial(
    pl.kernel,
    mesh=_mesh,
    out_type=jax.ShapeDtypeStruct((B * S * D,), jnp.float32),
    scratch_types=[
        pltpu.VMEM((NBUF, CD), jnp.float32),
        pltpu.VMEM((NBUF, B, CD), jnp.float32),
        pltpu.SemaphoreType.DMA((NBUF,)),
        pltpu.SemaphoreType.DMA((NBUF,)),
    ],
)
def _sc_add(x_hbm, pos_hbm, out_hbm, pos_v, xb_v, ld_sem, st_sem):
    wid = lax.axis_index("s") * NC + lax.axis_index("c")
    pos_row0 = wid * PRW

    def start_load(i, s):
        p_off = (pos_row0 + i * C) * D
        pltpu.async_copy(pos_hbm.at[pl.ds(p_off, CD)], pos_v.at[s], ld_sem.at[s])
        for b in range(B):
            pltpu.async_copy(
                x_hbm.at[pl.ds(b * S * D + p_off, CD)], xb_v.at[s, b],
                ld_sem.at[s])

    def wait_load(s):
        pltpu.make_async_copy(
            pos_hbm.at[pl.ds(0, CD)], pos_v.at[s], ld_sem.at[s]).wait()
        for b in range(B):
            pltpu.make_async_copy(
                x_hbm.at[pl.ds(0, CD)], xb_v.at[s, b], ld_sem.at[s]).wait()

    def start_store(i, s):
        p_off = (pos_row0 + i * C) * D
        for b in range(B):
            pltpu.async_copy(
                xb_v.at[s, b], out_hbm.at[pl.ds(b * S * D + p_off, CD)],
                st_sem.at[s])

    def wait_store(s):
        for b in range(B):
            pltpu.make_async_copy(
                xb_v.at[s, b], out_hbm.at[pl.ds(0, CD)], st_sem.at[s]).wait()

    def compute(s):
        def add_body(k, carry):
            base = k * (L * UNROLL)
            for j in range(UNROLL):
                sl = pl.ds(base + j * L, L)
                pv = pos_v[s, sl]
                for b in range(B):
                    xb_v[s, b, sl] = xb_v[s, b, sl] + pv
            return carry

        lax.fori_loop(0, CD // (L * UNROLL), add_body, 0)

    start_load(0, 0)

    def chunk_body(i, carry):
        s = lax.rem(i, NBUF)
        sn = lax.rem(i + 1, NBUF)

        @pl.when(i >= NBUF - 1)
        def _():
            wait_store(sn)

        @pl.when(i + 1 < NCHUNK)
        def _():
            start_load(i + 1, sn)

        wait_load(s)
        compute(s)
        start_store(i, s)
        return carry

    lax.fori_loop(0, NCHUNK, chunk_body, 0)
    wait_store((NCHUNK - 2) % NBUF)
    wait_store((NCHUNK - 1) % NBUF)


def kernel(x, pos_emb):
    xf = x.reshape(B * S * D)
    pf = pos_emb.reshape(S * D)
    out = _sc_add(xf, pf)
    return out.reshape(B, S, D)


# speedup vs baseline: 1.4302x; 1.4302x over previous
import functools

import jax
import jax.numpy as jnp
from jax import lax
from jax.experimental import pallas as pl
from jax.experimental.pallas import tpu as pltpu
from jax.experimental.pallas import tpu_sc as plsc

B = 4
S = 2048
D = 2048
NC = 2
NS = 16
NW = NC * NS
PRW = S // NW
C = 4
NCHUNK = PRW // C
L = 16
UNROLL = 8
NBUF = 3

_mesh = plsc.VectorSubcoreMesh(core_axis_name="c", subcore_axis_name="s")


@functools.partial(
    pl.kernel,
    mesh=_mesh,
    out_type=jax.ShapeDtypeStruct((B, S, D), jnp.float32),
    scratch_types=[
        pltpu.VMEM((NBUF, C, D), jnp.float32),
        pltpu.VMEM((NBUF, B, C, D), jnp.float32),
        pltpu.SemaphoreType.DMA((NBUF,)),
        pltpu.SemaphoreType.DMA((NBUF,)),
    ],
)
def _sc_add(x_hbm, pos_hbm, out_hbm, pos_v, xb_v, ld_sem, st_sem):
    wid = lax.axis_index("s") * NC + lax.axis_index("c")
    pos_row0 = wid * PRW

    def start_load(i, s):
        r = pos_row0 + i * C
        pltpu.async_copy(pos_hbm.at[pl.ds(r, C), :], pos_v.at[s], ld_sem.at[s])
        for b in range(B):
            pltpu.async_copy(
                x_hbm.at[b, pl.ds(r, C), :], xb_v.at[s, b], ld_sem.at[s])

    def wait_load(s):
        pltpu.make_async_copy(
            pos_hbm.at[pl.ds(0, C), :], pos_v.at[s], ld_sem.at[s]).wait()
        for b in range(B):
            pltpu.make_async_copy(
                x_hbm.at[0, pl.ds(0, C), :], xb_v.at[s, b], ld_sem.at[s]).wait()

    def start_store(i, s):
        r = pos_row0 + i * C
        for b in range(B):
            pltpu.async_copy(
                xb_v.at[s, b], out_hbm.at[b, pl.ds(r, C), :], st_sem.at[s])

    def wait_store(s):
        for b in range(B):
            pltpu.make_async_copy(
                xb_v.at[s, b], out_hbm.at[0, pl.ds(0, C), :],
                st_sem.at[s]).wait()

    def compute(s):
        def add_body(k, carry):
            base = k * (L * UNROLL)
            for row in range(C):
                for j in range(UNROLL):
                    sl = pl.ds(base + j * L, L)
                    pv = pos_v[s, row, sl]
                    for b in range(B):
                        xb_v[s, b, row, sl] = xb_v[s, b, row, sl] + pv
            return carry

        lax.fori_loop(0, D // (L * UNROLL), add_body, 0)

    start_load(0, 0)

    def chunk_body(i, carry):
        s = lax.rem(i, NBUF)
        sn = lax.rem(i + 1, NBUF)

        @pl.when(i >= NBUF - 1)
        def _():
            wait_store(sn)

        @pl.when(i + 1 < NCHUNK)
        def _():
            start_load(i + 1, sn)

        wait_load(s)
        compute(s)
        start_store(i, s)
        return carry

    lax.fori_loop(0, NCHUNK, chunk_body, 0)
    wait_store((NCHUNK - 2) % NBUF)
    wait_store((NCHUNK - 1) % NBUF)


def kernel(x, pos_emb):
    return _sc_add(x, pos_emb)
